# Initial kernel scaffold; baseline (speedup 1.0000x reference)
import jax
import jax.numpy as jnp
from jax.experimental import pallas as pl


def kernel(xyz, f, xyz_sampled, f_sampled):
    raise NotImplementedError("write your pallas kernel here")


# speedup vs baseline: 9.9511x; 9.9511x over previous
import functools

import jax
import jax.numpy as jnp
from jax import lax
from jax.experimental import pallas as pl
from jax.experimental.pallas import tpu as pltpu
from jax.experimental.pallas import tpu_sc as plsc

KNN = 32
S_BLK = 128

_NC = 2
_NS = 16
_NW = _NC * _NS


def _topk_body(q_ref, x_ref, qn2_ref, xn2_ref, out_ref, *, n_pts):
    b = pl.program_id(0)
    q = q_ref[0]
    x = x_ref[0]
    qx = lax.dot_general(
        q.astype(jnp.bfloat16), x.astype(jnp.bfloat16),
        (((1,), (0,)), ((), ())),
        preferred_element_type=jnp.float32)
    d = (qn2_ref[0] - 2.0 * qx) + xn2_ref[0]
    iota = lax.broadcasted_iota(jnp.int32, d.shape, 1)
    inf = jnp.float32(jnp.inf)
    cols = []
    for _ in range(KNN):
        m = jnp.min(d, axis=1, keepdims=True)
        am = jnp.min(jnp.where(d == m, iota, n_pts), axis=1, keepdims=True)
        cols.append(am)
        d = jnp.where(iota == am, inf, d)
    idx = jnp.concatenate(cols, axis=1)
    out_ref[0] = idx + b * n_pts


def _topk_call(q, xt, qn2, xn2):
    bsz, _, n_pts = xt.shape
    s = q.shape[1]
    return pl.pallas_call(
        functools.partial(_topk_body, n_pts=n_pts),
        grid=(bsz, s // S_BLK),
        in_specs=[
            pl.BlockSpec((1, S_BLK, 3), lambda b, i: (b, i, 0)),
            pl.BlockSpec((1, 3, n_pts), lambda b, i: (b, 0, 0)),
            pl.BlockSpec((1, S_BLK, 1), lambda b, i: (b, i, 0)),
            pl.BlockSpec((1, 1, n_pts), lambda b, i: (b, 0, 0)),
        ],
        out_specs=pl.BlockSpec((1, S_BLK, KNN), lambda b, i: (b, i, 0)),
        out_shape=jax.ShapeDtypeStruct((bsz, s, KNN), jnp.int32),
    )(q, xt, qn2, xn2)


_GCHUNK = 512


def _gather_body(gidx_hbm, f_hbm, x_hbm, fg_hbm, xg_hbm,
                 idxv, fv, xv, sem_f, sem_x, *, rows_per_w, fdim, xdim):
    wid = lax.axis_index("s") * _NC + lax.axis_index("c")
    base = wid * rows_per_w

    def chunk(i, carry):
        off = base + i * _GCHUNK
        pltpu.sync_copy(gidx_hbm.at[pl.ds(off, _GCHUNK)], idxv)
        cp_f = pltpu.async_copy(f_hbm.at[idxv], fv, sem_f)
        cp_x = pltpu.async_copy(x_hbm.at[idxv], xv, sem_x)
        cp_f.wait()
        cp_x.wait()
        pltpu.sync_copy(fv, fg_hbm.at[pl.ds(off, _GCHUNK)])
        pltpu.sync_copy(xv, xg_hbm.at[pl.ds(off, _GCHUNK)])
        return carry

    lax.fori_loop(0, rows_per_w // _GCHUNK, chunk, 0)


def _gather_call(gidx, f2, x2):
    total = gidx.shape[0]
    fdim = f2.shape[1]
    xdim = x2.shape[1]
    rows_per_w = total // _NW
    mesh = plsc.VectorSubcoreMesh(core_axis_name="c", subcore_axis_name="s")
    body = functools.partial(
        _gather_body, rows_per_w=rows_per_w, fdim=fdim, xdim=xdim)
    return pl.kernel(
        body,
        out_type=[
            jax.ShapeDtypeStruct((total, fdim), jnp.float32),
            jax.ShapeDtypeStruct((total, xdim), jnp.float32),
        ],
        mesh=mesh,
        scratch_types=[
            pltpu.VMEM((_GCHUNK,), jnp.int32),
            pltpu.VMEM((_GCHUNK, fdim), jnp.float32),
            pltpu.VMEM((_GCHUNK, xdim), jnp.float32),
            pltpu.SemaphoreType.DMA,
            pltpu.SemaphoreType.DMA,
        ],
        compiler_params=pltpu.CompilerParams(use_tc_tiling_on_sc=False),
    )(gidx, f2, x2)


def kernel(xyz, f, xyz_sampled, f_sampled):
    bsz, n_pts, _ = xyz.shape
    s = xyz_sampled.shape[1]
    fdim = f.shape[2]

    xt = xyz.transpose(0, 2, 1)
    qn2 = jnp.sum(xyz_sampled ** 2, axis=-1, keepdims=True)
    xn2 = jnp.sum(xyz ** 2, axis=-1)[:, None, :]
    gidx = _topk_call(xyz_sampled, xt, qn2, xn2)

    f2 = f.reshape(bsz * n_pts, fdim)
    x2 = jnp.pad(xyz, ((0, 0), (0, 0), (0, 13))).reshape(bsz * n_pts, 16)
    fg, xg = _gather_call(gidx.reshape(-1), f2, x2)

    xyz_grouped = xg.reshape(bsz, s, KNN, 16)[..., :3]
    f_grouped = fg.reshape(bsz, s, KNN, fdim)
    return (xyz_grouped, f_grouped)


# speedup vs baseline: 22.3388x; 2.2449x over previous
import functools

import jax
import jax.numpy as jnp
from jax import lax
from jax.experimental import pallas as pl
from jax.experimental.pallas import tpu as pltpu
from jax.experimental.pallas import tpu_sc as plsc

KNN = 32
S_BLK = 128
NSEG = 128
DEPTH = 6

_NC = 2
_NS = 16
_NW = _NC * _NS


def _topk_body(x_ref, qt_ref, qn2_ref, xn2_ref, out_ref, *, n_pts):
    b = pl.program_id(0)
    x = x_ref[0]
    qt = qt_ref[0]
    segr = n_pts // NSEG
    qx = lax.dot_general(
        x.astype(jnp.bfloat16), qt.astype(jnp.bfloat16),
        (((1,), (0,)), ((), ())),
        preferred_element_type=jnp.float32)
    dt = (qn2_ref[0] - 2.0 * qx) + xn2_ref[0]

    inf = jnp.float32(jnp.inf)
    dr = dt.reshape(NSEG, segr, S_BLK)
    ig = (lax.broadcasted_iota(jnp.int32, (NSEG, segr, S_BLK), 0) * segr
          + lax.broadcasted_iota(jnp.int32, (NSEG, segr, S_BLK), 1))

    vs, js = [], []
    for t in range(DEPTH):
        mv = jnp.min(dr, axis=1, keepdims=True)
        cand = jnp.where(dr == mv, ig, n_pts)
        mi = jnp.min(cand, axis=1, keepdims=True)
        vs.append(mv[:, 0, :])
        js.append(mi[:, 0, :])
        if t < DEPTH - 1:
            dr = jnp.where(cand == mi, inf, dr)

    segi = lax.broadcasted_iota(jnp.int32, (NSEG, S_BLK), 0)
    pops = jnp.zeros((NSEG, S_BLK), jnp.int32)
    rows = []
    for _ in range(KNN):
        m = jnp.min(vs[0], axis=0, keepdims=True)
        sc = jnp.where(vs[0] == m, segi, NSEG)
        s_star = jnp.min(sc, axis=0, keepdims=True)
        oh = segi == s_star
        rows.append(jnp.sum(jnp.where(oh, js[0], 0), axis=0, keepdims=True))
        pops = pops + oh.astype(jnp.int32)
        for j in range(DEPTH - 1):
            vs[j] = jnp.where(oh, vs[j + 1], vs[j])
            js[j] = jnp.where(oh, js[j + 1], js[j])
        vs[DEPTH - 1] = jnp.where(oh, inf, vs[DEPTH - 1])
    idx_fast = jnp.concatenate(rows, axis=0)

    overflow = jnp.any(pops >= DEPTH)

    def slow():
        iota_full = lax.broadcasted_iota(jnp.int32, (n_pts, S_BLK), 0)
        dd = dt
        out = []
        for _ in range(KNN):
            m = jnp.min(dd, axis=0, keepdims=True)
            cand = jnp.where(dd == m, iota_full, n_pts)
            am = jnp.min(cand, axis=0, keepdims=True)
            out.append(am)
            dd = jnp.where(cand == am, inf, dd)
        return jnp.concatenate(out, axis=0)

    idx = lax.cond(overflow, slow, lambda: idx_fast)
    out_ref[0] = idx + b * n_pts


def _topk_call(qt, x, qn2r, xn2c):
    bsz, n_pts, _ = x.shape
    s = qt.shape[2]
    return pl.pallas_call(
        functools.partial(_topk_body, n_pts=n_pts),
        grid=(bsz, s // S_BLK),
        in_specs=[
            pl.BlockSpec((1, n_pts, 3), lambda b, i: (b, 0, 0)),
            pl.BlockSpec((1, 3, S_BLK), lambda b, i: (b, 0, i)),
            pl.BlockSpec((1, 1, S_BLK), lambda b, i: (b, 0, i)),
            pl.BlockSpec((1, n_pts, 1), lambda b, i: (b, 0, 0)),
        ],
        out_specs=pl.BlockSpec((1, KNN, S_BLK), lambda b, i: (b, 0, i)),
        out_shape=jax.ShapeDtypeStruct((bsz, KNN, s), jnp.int32),
    )(x, qt, qn2r, xn2c)


_GCHUNK = 512


def _gather_body(gidx_hbm, f_hbm, x_hbm, fg_hbm, xg_hbm,
                 idxv, fv, xv, sem_f, sem_x, *, rows_per_w, fdim, xdim):
    wid = lax.axis_index("s") * _NC + lax.axis_index("c")
    base = wid * rows_per_w

    def chunk(i, carry):
        off = base + i * _GCHUNK
        pltpu.sync_copy(gidx_hbm.at[pl.ds(off, _GCHUNK)], idxv)
        cp_f = pltpu.async_copy(f_hbm.at[idxv], fv, sem_f)
        cp_x = pltpu.async_copy(x_hbm.at[idxv], xv, sem_x)
        cp_f.wait()
        cp_x.wait()
        pltpu.sync_copy(fv, fg_hbm.at[pl.ds(off, _GCHUNK)])
        pltpu.sync_copy(xv, xg_hbm.at[pl.ds(off, _GCHUNK)])
        return carry

    lax.fori_loop(0, rows_per_w // _GCHUNK, chunk, 0)


def _gather_call(gidx, f2, x2):
    total = gidx.shape[0]
    fdim = f2.shape[1]
    xdim = x2.shape[1]
    rows_per_w = total // _NW
    mesh = plsc.VectorSubcoreMesh(core_axis_name="c", subcore_axis_name="s")
    body = functools.partial(
        _gather_body, rows_per_w=rows_per_w, fdim=fdim, xdim=xdim)
    return pl.kernel(
        body,
        out_type=[
            jax.ShapeDtypeStruct((total, fdim), jnp.float32),
            jax.ShapeDtypeStruct((total, xdim), jnp.float32),
        ],
        mesh=mesh,
        scratch_types=[
            pltpu.VMEM((_GCHUNK,), jnp.int32),
            pltpu.VMEM((_GCHUNK, fdim), jnp.float32),
            pltpu.VMEM((_GCHUNK, xdim), jnp.float32),
            pltpu.SemaphoreType.DMA,
            pltpu.SemaphoreType.DMA,
        ],
        compiler_params=pltpu.CompilerParams(use_tc_tiling_on_sc=False),
    )(gidx, f2, x2)


def kernel(xyz, f, xyz_sampled, f_sampled):
    bsz, n_pts, _ = xyz.shape
    s = xyz_sampled.shape[1]
    fdim = f.shape[2]

    qt = xyz_sampled.transpose(0, 2, 1)
    qn2r = jnp.sum(xyz_sampled ** 2, axis=-1)[:, None, :]
    xn2c = jnp.sum(xyz ** 2, axis=-1)[:, :, None]
    gidx = _topk_call(qt, xyz, qn2r, xn2c)
    gidx = gidx.transpose(0, 2, 1)

    f2 = f.reshape(bsz * n_pts, fdim)
    x2 = jnp.pad(xyz, ((0, 0), (0, 0), (0, 13))).reshape(bsz * n_pts, 16)
    fg, xg = _gather_call(gidx.reshape(-1), f2, x2)

    xyz_grouped = xg.reshape(bsz, s, KNN, 16)[..., :3]
    f_grouped = fg.reshape(bsz, s, KNN, fdim)
    return (xyz_grouped, f_grouped)


# speedup vs baseline: 22.4886x; 1.0067x over previous
import functools

import jax
import jax.numpy as jnp
from jax import lax
from jax.experimental import pallas as pl
from jax.experimental.pallas import tpu as pltpu
from jax.experimental.pallas import tpu_sc as plsc

KNN = 32
S_BLK = 128
NSEG = 128
DEPTH = 6

_NC = 2
_NS = 16
_NW = _NC * _NS


def _topk_body(x_ref, qt_ref, qn2_ref, xn2_ref, out_ref, *, n_pts):
    b = pl.program_id(0)
    x = x_ref[0]
    qt = qt_ref[0]
    segr = n_pts // NSEG
    qx = lax.dot_general(
        x.astype(jnp.bfloat16), qt.astype(jnp.bfloat16),
        (((1,), (0,)), ((), ())),
        preferred_element_type=jnp.float32)
    dt = (qn2_ref[0] - 2.0 * qx) + xn2_ref[0]

    inf = jnp.float32(jnp.inf)
    dr = dt.reshape(NSEG, segr, S_BLK)
    ig = (lax.broadcasted_iota(jnp.int32, (NSEG, segr, S_BLK), 0) * segr
          + lax.broadcasted_iota(jnp.int32, (NSEG, segr, S_BLK), 1))

    vs, js = [], []
    for t in range(DEPTH):
        mv = jnp.min(dr, axis=1, keepdims=True)
        cand = jnp.where(dr == mv, ig, n_pts)
        mi = jnp.min(cand, axis=1, keepdims=True)
        vs.append(mv[:, 0, :])
        js.append(mi[:, 0, :])
        if t < DEPTH - 1:
            dr = jnp.where(cand == mi, inf, dr)

    segi = lax.broadcasted_iota(jnp.int32, (NSEG, S_BLK), 0)
    pops = jnp.zeros((NSEG, S_BLK), jnp.int32)
    rows = []
    for _ in range(KNN):
        m = jnp.min(vs[0], axis=0, keepdims=True)
        sc = jnp.where(vs[0] == m, segi, NSEG)
        s_star = jnp.min(sc, axis=0, keepdims=True)
        oh = segi == s_star
        rows.append(jnp.sum(jnp.where(oh, js[0], 0), axis=0, keepdims=True))
        pops = pops + oh.astype(jnp.int32)
        for j in range(DEPTH - 1):
            vs[j] = jnp.where(oh, vs[j + 1], vs[j])
            js[j] = jnp.where(oh, js[j + 1], js[j])
        vs[DEPTH - 1] = jnp.where(oh, inf, vs[DEPTH - 1])
    idx_fast = jnp.concatenate(rows, axis=0)

    overflow = jnp.any(pops >= DEPTH)

    def slow():
        iota_full = lax.broadcasted_iota(jnp.int32, (n_pts, S_BLK), 0)
        dd = dt
        out = []
        for _ in range(KNN):
            m = jnp.min(dd, axis=0, keepdims=True)
            cand = jnp.where(dd == m, iota_full, n_pts)
            am = jnp.min(cand, axis=0, keepdims=True)
            out.append(am)
            dd = jnp.where(cand == am, inf, dd)
        return jnp.concatenate(out, axis=0)

    idx = lax.cond(overflow, slow, lambda: idx_fast)
    out_ref[0] = idx + b * n_pts


def _topk_call(qt, x, qn2r, xn2c):
    bsz, n_pts, _ = x.shape
    s = qt.shape[2]
    return pl.pallas_call(
        functools.partial(_topk_body, n_pts=n_pts),
        grid=(bsz, s // S_BLK),
        in_specs=[
            pl.BlockSpec((1, n_pts, 3), lambda b, i: (b, 0, 0)),
            pl.BlockSpec((1, 3, S_BLK), lambda b, i: (b, 0, i)),
            pl.BlockSpec((1, 1, S_BLK), lambda b, i: (b, 0, i)),
            pl.BlockSpec((1, n_pts, 1), lambda b, i: (b, 0, 0)),
        ],
        out_specs=pl.BlockSpec((1, KNN, S_BLK), lambda b, i: (b, 0, i)),
        out_shape=jax.ShapeDtypeStruct((bsz, KNN, s), jnp.int32),
    )(x, qt, qn2r, xn2c)


_GCHUNK = 1024


def _gather_body(gidx_hbm, f_hbm, x_hbm, fg_hbm, xg_hbm,
                 idxv, fv, xv, sem_f, sem_x, *, rows_per_w, fdim, xdim):
    wid = lax.axis_index("s") * _NC + lax.axis_index("c")
    base = wid * rows_per_w

    def chunk(i, carry):
        off = base + i * _GCHUNK
        pltpu.sync_copy(gidx_hbm.at[pl.ds(off, _GCHUNK)], idxv)
        cp_f = pltpu.async_copy(f_hbm.at[idxv], fv, sem_f)
        cp_x = pltpu.async_copy(x_hbm.at[idxv], xv, sem_x)
        cp_f.wait()
        cp_x.wait()
        pltpu.sync_copy(fv, fg_hbm.at[pl.ds(off, _GCHUNK)])
        pltpu.sync_copy(xv, xg_hbm.at[pl.ds(off, _GCHUNK)])
        return carry

    lax.fori_loop(0, rows_per_w // _GCHUNK, chunk, 0)


def _gather_call(gidx, f2, x2):
    total = gidx.shape[0]
    fdim = f2.shape[1]
    xdim = x2.shape[1]
    rows_per_w = total // _NW
    mesh = plsc.VectorSubcoreMesh(core_axis_name="c", subcore_axis_name="s")
    body = functools.partial(
        _gather_body, rows_per_w=rows_per_w, fdim=fdim, xdim=xdim)
    return pl.kernel(
        body,
        out_type=[
            jax.ShapeDtypeStruct((total, fdim), jnp.float32),
            jax.ShapeDtypeStruct((total, xdim), jnp.float32),
        ],
        mesh=mesh,
        scratch_types=[
            pltpu.VMEM((_GCHUNK,), jnp.int32),
            pltpu.VMEM((_GCHUNK, fdim), jnp.float32),
            pltpu.VMEM((_GCHUNK, xdim), jnp.float32),
            pltpu.SemaphoreType.DMA,
            pltpu.SemaphoreType.DMA,
        ],
        compiler_params=pltpu.CompilerParams(use_tc_tiling_on_sc=False),
    )(gidx, f2, x2)


def kernel(xyz, f, xyz_sampled, f_sampled):
    bsz, n_pts, _ = xyz.shape
    s = xyz_sampled.shape[1]
    fdim = f.shape[2]

    qt = xyz_sampled.transpose(0, 2, 1)
    qn2r = jnp.sum(xyz_sampled ** 2, axis=-1)[:, None, :]
    xn2c = jnp.sum(xyz ** 2, axis=-1)[:, :, None]
    gidx = _topk_call(qt, xyz, qn2r, xn2c)
    gidx = gidx.transpose(0, 2, 1)

    f2 = f.reshape(bsz * n_pts, fdim)
    x2 = jnp.pad(xyz, ((0, 0), (0, 0), (0, 13))).reshape(bsz * n_pts, 16)
    fg, xg = _gather_call(gidx.reshape(-1), f2, x2)

    xyz_grouped = xg.reshape(bsz, s, KNN, 16)[..., :3]
    f_grouped = fg.reshape(bsz, s, KNN, fdim)
    return (xyz_grouped, f_grouped)
